# baseline (speedup 1.0000x reference)
import functools

import jax
import jax.numpy as jnp
from jax import lax
from jax.experimental import pallas as pl
from jax.experimental.pallas import tpu as pltpu
from jax.experimental.pallas import tpu_sc as plsc

N = 16384
K = 2048
BN = 1024
NB = N // BN
NSUB = 32
RP = N // NSUB
CH = K // 16


def _prep_body(pt_ref, th_ref, c_ref, rd_ref, cb_ref):
    pt = pt_ref[...]
    th = th_ref[...]
    cos = jnp.cos(th)
    sin = jnp.sin(th)
    dx1 = pt[2:3, :] - pt[0:1, :]
    dy1 = pt[3:4, :] - pt[1:2, :]
    dx2 = pt[4:5, :] - pt[0:1, :]
    dy2 = pt[5:6, :] - pt[1:2, :]
    px1 = dx1 * cos + dy1 * sin
    py1 = dy1 * cos - dx1 * sin
    px2 = dx2 * cos + dy2 * sin
    py2 = dy2 * cos - dx2 * sin
    pn = dx1 * dx1 + dy1 * dy1 + dx2 * dx2 + dy2 * dy2
    zero3 = jnp.zeros((3, pt.shape[1]), jnp.float32)
    rd_ref[...] = jnp.concatenate([px1, py1, px2, py2, pn, zero3], axis=0)

    c = c_ref[...]
    cx1 = c[2:3, :]
    cy1 = c[3:4, :]
    cx2 = c[4:5, :]
    cy2 = c[5:6, :]
    e = (c[0:1, :] * c[0:1, :] + c[1:2, :] * c[1:2, :]
         + cx1 * cx1 + cy1 * cy1 + cx2 * cx2 + cy2 * cy2)
    zk3 = jnp.zeros((3, K), jnp.float32)
    cb_ref[...] = jnp.concatenate(
        [2.0 * cx1, 2.0 * cy1, 2.0 * cx2, 2.0 * cy2, e, zk3], axis=0)


def _tc_prep(traj_pos, traj_theta, map_token_sample_pt):
    pt = traj_pos.reshape(N, 6).T
    th = traj_theta.reshape(1, N)
    c = map_token_sample_pt.reshape(K, 6).T
    return pl.pallas_call(
        _prep_body,
        grid=(NB,),
        in_specs=[
            pl.BlockSpec((6, BN), lambda i: (0, i)),
            pl.BlockSpec((1, BN), lambda i: (0, i)),
            pl.BlockSpec((6, K), lambda i: (0, 0)),
        ],
        out_specs=[
            pl.BlockSpec((8, BN), lambda i: (0, i)),
            pl.BlockSpec((8, K), lambda i: (0, 0)),
        ],
        out_shape=[
            jax.ShapeDtypeStruct((8, N), jnp.float32),
            jax.ShapeDtypeStruct((8, K), jnp.float32),
        ],
    )(pt, th, c)


G = 4


def _sc_body(cb_hbm, rd_hbm, idx_hbm, md_hbm, cb_v, rd_v, idx_v, md_v):
    wid = lax.axis_index("s") * 2 + lax.axis_index("c")
    base = wid * RP
    pltpu.sync_copy(cb_hbm, cb_v)
    pltpu.sync_copy(rd_hbm.at[:, pl.ds(base, RP)], rd_v)
    kiota = lax.iota(jnp.int32, 16)
    lane0 = kiota == 0

    def macro_body(mb, _):
        rbase = mb * 16
        av1 = rd_v[0, pl.ds(rbase, 16)]
        av2 = rd_v[1, pl.ds(rbase, 16)]
        av3 = rd_v[2, pl.ds(rbase, 16)]
        av4 = rd_v[3, pl.ds(rbase, 16)]
        apn = rd_v[4, pl.ds(rbase, 16)]

        for sub in range(16 // G):
            s1, s2, s3, s4 = [], [], [], []
            for i in range(G):
                li = jnp.full((16,), sub * G + i, jnp.int32)
                s1.append(jnp.take_along_axis(av1, li, axis=0))
                s2.append(jnp.take_along_axis(av2, li, axis=0))
                s3.append(jnp.take_along_axis(av3, li, axis=0))
                s4.append(jnp.take_along_axis(av4, li, axis=0))

            def chunk_body(j, carry, s1=s1, s2=s2, s3=s3, s4=s4):
                best, bidx = carry
                o = j * 16
                c1 = cb_v[0, pl.ds(o, 16)]
                c2 = cb_v[1, pl.ds(o, 16)]
                c3 = cb_v[2, pl.ds(o, 16)]
                c4 = cb_v[3, pl.ds(o, 16)]
                ev = cb_v[4, pl.ds(o, 16)]
                kv = kiota + o
                nbest, nbidx = [], []
                for i in range(G):
                    d = ev - (c1 * s1[i] + c2 * s2[i] + c3 * s3[i] + c4 * s4[i])
                    lt = d < best[i]
                    nbest.append(jnp.where(lt, d, best[i]))
                    nbidx.append(jnp.where(lt, kv, bidx[i]))
                return tuple(nbest), tuple(nbidx)

            best0 = tuple(jnp.full((16,), jnp.inf, jnp.float32) for _ in range(G))
            bidx0 = tuple(jnp.zeros((16,), jnp.int32) for _ in range(G))
            best, bidx = lax.fori_loop(0, CH, chunk_body, (best0, bidx0))

            for i in range(G):
                mv = jnp.min(best[i])
                bi = jnp.min(jnp.where(best[i] == mv, bidx[i], jnp.int32(K)))
                r = rbase + sub * G + i
                rv = jnp.full((16,), r, jnp.int32)
                plsc.store_scatter(idx_v, [rv], jnp.full((16,), bi, jnp.int32),
                                   mask=lane0)
                plsc.store_scatter(md_v, [rv], jnp.full((16,), mv + apn[sub * G + i],
                                                        jnp.float32), mask=lane0)
        return 0

    lax.fori_loop(0, RP // 16, macro_body, 0)
    pltpu.sync_copy(idx_v, idx_hbm.at[pl.ds(base, RP)])
    pltpu.sync_copy(md_v, md_hbm.at[pl.ds(base, RP)])


_sc_call = functools.partial(
    pl.kernel,
    mesh=plsc.VectorSubcoreMesh(core_axis_name="c", subcore_axis_name="s"),
    compiler_params=pltpu.CompilerParams(needs_layout_passes=False),
    out_type=[
        jax.ShapeDtypeStruct((N,), jnp.int32),
        jax.ShapeDtypeStruct((N,), jnp.float32),
    ],
    scratch_types=[
        pltpu.VMEM((8, K), jnp.float32),
        pltpu.VMEM((8, RP), jnp.float32),
        pltpu.VMEM((RP,), jnp.int32),
        pltpu.VMEM((RP,), jnp.float32),
    ],
)(_sc_body)


@jax.jit
def kernel(traj_pos, traj_theta, map_token_sample_pt):
    rd, cb = _tc_prep(traj_pos, traj_theta, map_token_sample_pt)
    idx, md = _sc_call(cb, rd)
    return (traj_pos[:, 0], traj_theta, idx, md)
